# Initial kernel scaffold; baseline (speedup 1.0000x reference)
import jax
import jax.numpy as jnp
from jax.experimental import pallas as pl


def kernel(x, edge_index, edge_weight, Wz, bz, Wr, br, Wh, bh, Lz_w, Lz_b, Lr_w, Lr_b, Lh_w, Lh_b, head_w, head_b):
    raise NotImplementedError("write your pallas kernel here")


# speedup vs baseline: 16.7502x; 16.7502x over previous
import functools

import jax
import jax.numpy as jnp
from jax import lax
from jax.experimental import pallas as pl
from jax.experimental.pallas import tpu as pltpu
from jax.experimental.pallas import tpu_sc as plsc

N_NODES = 10000
D = 128
NPAD = 10240
ROWS_PER_TILE = NPAD // 16
NC = 2
NS = 16
CHUNK = 128
NCH = 80
EPT = NCH * CHUNK
EPAD = NC * NS * EPT


def _rsqrt16(d):
    y = jnp.full((16,), 1.0, jnp.float32)
    for j in range(1, 21):
        y = jnp.where(d >= jnp.float32(2.0 ** j), jnp.float32(2.0 ** (-0.5 * (j + 1))), y)
    for _ in range(5):
        y = y * (1.5 - 0.5 * d * y * y)
    return y


def _sc_body(x_hbm, srcb, dstb, ewb, pout, dinv_out,
             src_v, dst_v, ew_v, dinv_v, rows_v, norm_v, degbuf,
             p_sh, deg_sh, sem0):
    c = lax.axis_index("c")
    s = lax.axis_index("s")
    w = c * NS + s
    base = s * ROWS_PER_TILE

    def fill_deg(k, _):
        degbuf[pl.ds(k * 16, 16)] = jnp.full((16,), 1.0, jnp.float32)
        return _
    lax.fori_loop(0, ROWS_PER_TILE // 16, fill_deg, None)
    pltpu.sync_copy(degbuf, deg_sh.at[pl.ds(base, ROWS_PER_TILE)])

    def zero_row(r, _):
        for cc in range(D // 16):
            rows_v[r, pl.ds(cc * 16, 16)] = jnp.zeros((16,), jnp.float32)
        return _
    lax.fori_loop(0, CHUNK, zero_row, None)
    for t in range(ROWS_PER_TILE // CHUNK):
        pltpu.sync_copy(rows_v, p_sh.at[pl.ds(base + t * CHUNK, CHUNK)])
    plsc.subcore_barrier()

    for t in range(2):
        blk = s + t * NS
        for g in range(2):
            pltpu.sync_copy(dstb.at[blk, pl.ds(g * (NCH // 2), NCH // 2)], dst_v)
            pltpu.sync_copy(ewb.at[blk, pl.ds(g * (NCH // 2), NCH // 2)], ew_v)

            def deg_chunk(j, _):
                pltpu.sync_copy(ew_v.at[j], deg_sh.at[dst_v.at[j]], add=True)
                return _
            lax.fori_loop(0, NCH // 2, deg_chunk, None)
    plsc.subcore_barrier()

    pltpu.sync_copy(deg_sh.at[pl.ds(base, ROWS_PER_TILE)], degbuf)

    def rs_step(k, _):
        sl = pl.ds(k * 16, 16)
        degbuf[sl] = _rsqrt16(degbuf[sl])
        return _
    lax.fori_loop(0, ROWS_PER_TILE // 16, rs_step, None)
    pltpu.sync_copy(degbuf, deg_sh.at[pl.ds(base, ROWS_PER_TILE)])

    @pl.when(c == 0)
    def _():
        pltpu.sync_copy(degbuf, dinv_out.at[pl.ds(base, ROWS_PER_TILE)])
    plsc.subcore_barrier()
    pltpu.sync_copy(deg_sh, dinv_v)

    for g in range(2):
        pltpu.sync_copy(srcb.at[w, pl.ds(g * (NCH // 2), NCH // 2)], src_v)
        pltpu.sync_copy(dstb.at[w, pl.ds(g * (NCH // 2), NCH // 2)], dst_v)
        pltpu.sync_copy(ewb.at[w, pl.ds(g * (NCH // 2), NCH // 2)], ew_v)

        def chunk_body(j, _):
            pltpu.async_copy(x_hbm.at[src_v.at[j]], rows_v, sem0).wait()
            for cc in range(CHUNK // 16):
                sl = pl.ds(cc * 16, 16)
                ds_ = plsc.load_gather(dinv_v, [src_v[j, sl]])
                dd_ = plsc.load_gather(dinv_v, [dst_v[j, sl]])
                norm_v[sl] = ds_ * ew_v[j, sl] * dd_

            def scale_row(r, _2):
                nsp = plsc.load_gather(norm_v, [jnp.full((16,), r, jnp.int32)])
                for cc in range(D // 16):
                    sl = pl.ds(cc * 16, 16)
                    rows_v[r, sl] = rows_v[r, sl] * nsp
                return _2
            lax.fori_loop(0, CHUNK, scale_row, None)
            pltpu.sync_copy(rows_v, p_sh.at[dst_v.at[j]], add=True)
            return _
        lax.fori_loop(0, NCH // 2, chunk_body, None)
    plsc.subcore_barrier()

    pltpu.sync_copy(p_sh.at[pl.ds(base, ROWS_PER_TILE)],
                    pout.at[c, pl.ds(base, ROWS_PER_TILE)])


def _sc_propagate(x, srcb, dstb, ewb):
    mesh = plsc.VectorSubcoreMesh(core_axis_name="c", subcore_axis_name="s")
    f = pl.kernel(
        _sc_body,
        out_type=[
            jax.ShapeDtypeStruct((NC, NPAD, D), jnp.float32),
            jax.ShapeDtypeStruct((NPAD,), jnp.float32),
        ],
        mesh=mesh,
        scratch_types=[
            pltpu.VMEM((NCH // 2, CHUNK), jnp.int32),
            pltpu.VMEM((NCH // 2, CHUNK), jnp.int32),
            pltpu.VMEM((NCH // 2, CHUNK), jnp.float32),
            pltpu.VMEM((NPAD,), jnp.float32),
            pltpu.VMEM((CHUNK, D), jnp.float32),
            pltpu.VMEM((CHUNK,), jnp.float32),
            pltpu.VMEM((ROWS_PER_TILE,), jnp.float32),
            pltpu.VMEM_SHARED((NPAD, D), jnp.float32),
            pltpu.VMEM_SHARED((NPAD,), jnp.float32),
            pltpu.SemaphoreType.DMA,
        ],
        compiler_params=pltpu.CompilerParams(needs_layout_passes=False),
    )
    return f(x, srcb, dstb, ewb)


def _dense_body(x_ref, p0_ref, p1_ref, dv_ref, wz_ref, bz_ref, wh_ref, bh_ref,
                lzw_ref, lzb_ref, lhw_ref, lhb_ref, hw_ref, hb_ref, o_ref):
    dv = dv_ref[...]
    P = p0_ref[...] + p1_ref[...] + (dv * dv) * x_ref[...]
    f32 = jnp.float32
    mz = jnp.dot(wz_ref[...], lzw_ref[...], preferred_element_type=f32)
    cz = jnp.dot(bz_ref[...], lzw_ref[...], preferred_element_type=f32) + lzb_ref[...]
    z = jax.nn.sigmoid(jnp.dot(P, mz, preferred_element_type=f32) + cz)
    mh = jnp.dot(wh_ref[...], lhw_ref[...], preferred_element_type=f32)
    ch = jnp.dot(bh_ref[...], lhw_ref[...], preferred_element_type=f32) + lhb_ref[...]
    ht = jnp.tanh(jnp.dot(P, mh, preferred_element_type=f32) + ch)
    o_ref[...] = jnp.dot((1.0 - z) * ht, hw_ref[...], preferred_element_type=f32) + hb_ref[...]


def _dense(xp, p0, p1, dv2, Wz, bz, Wh, bh, Lzw1, Lzb, Lhw1, Lhb, head_w, head_b):
    blk = 1280
    grid = (NPAD // blk,)
    row = lambda i: (i, 0)
    fix = lambda i: (0, 0)
    return pl.pallas_call(
        _dense_body,
        grid=grid,
        in_specs=[
            pl.BlockSpec((blk, D), row),
            pl.BlockSpec((blk, D), row),
            pl.BlockSpec((blk, D), row),
            pl.BlockSpec((blk, 1), row),
            pl.BlockSpec((D, D), fix),
            pl.BlockSpec((1, D), fix),
            pl.BlockSpec((D, D), fix),
            pl.BlockSpec((1, D), fix),
            pl.BlockSpec((D, D), fix),
            pl.BlockSpec((1, D), fix),
            pl.BlockSpec((D, D), fix),
            pl.BlockSpec((1, D), fix),
            pl.BlockSpec((D, 1), fix),
            pl.BlockSpec((1, 1), fix),
        ],
        out_specs=pl.BlockSpec((blk, 1), row),
        out_shape=jax.ShapeDtypeStruct((NPAD, 1), jnp.float32),
    )(xp, p0, p1, dv2, Wz, bz, Wh, bh, Lzw1, Lzb, Lhw1, Lhb, head_w, head_b)


def kernel(x, edge_index, edge_weight, Wz, bz, Wr, br, Wh, bh,
           Lz_w, Lz_b, Lr_w, Lr_b, Lh_w, Lh_b, head_w, head_b):
    E = edge_index.shape[1]
    pad = EPAD - E
    src = jnp.concatenate([edge_index[0], jnp.zeros((pad,), jnp.int32)])
    dst = jnp.concatenate([edge_index[1], jnp.zeros((pad,), jnp.int32)])
    ew = jnp.concatenate([edge_weight, jnp.zeros((pad,), jnp.float32)])
    srcb = src.reshape(NC * NS, NCH, CHUNK)
    dstb = dst.reshape(NC * NS, NCH, CHUNK)
    ewb = ew.reshape(NC * NS, NCH, CHUNK)

    pout, dinv = _sc_propagate(x, srcb, dstb, ewb)

    xp = jnp.zeros((NPAD, D), jnp.float32).at[:N_NODES].set(x)
    out = _dense(
        xp, pout[0], pout[1], dinv[:, None],
        Wz, bz[None, :], Wh, bh[None, :],
        Lz_w[:D], Lz_b[None, :], Lh_w[:D], Lh_b[None, :],
        head_w, head_b[None, :],
    )
    return out[:N_NODES, 0]


# speedup vs baseline: 20.5937x; 1.2295x over previous
import jax
import jax.numpy as jnp
from jax import lax
from jax.experimental import pallas as pl
from jax.experimental.pallas import tpu as pltpu
from jax.experimental.pallas import tpu_sc as plsc

N_NODES = 10000
D = 128
NPAD = 10240
ROWS_PER_TILE = NPAD // 16
DR = NPAD // 16
NC = 2
NS = 16
CHUNK = 64
NCH = 160
GRP = NCH // 4
EPT = NCH * CHUNK
EPAD = NC * NS * EPT


def _rsqrt16(d):
    y = jnp.full((16,), 1.0, jnp.float32)
    for j in range(1, 21):
        y = jnp.where(d >= jnp.float32(2.0 ** j), jnp.float32(2.0 ** (-0.5 * (j + 1))), y)
    for _ in range(5):
        y = y * (1.5 - 0.5 * d * y * y)
    return y


def _sc_body(x_hbm, srcb, dstb, ewb, pout, dinv_out,
             src_v, dst_v, ew_v, dinv_v, rows0, rows1, norm_v, degbuf,
             p_sh, deg_sh, sg0, sg1):
    c = lax.axis_index("c")
    s = lax.axis_index("s")
    w = c * NS + s
    base = s * ROWS_PER_TILE

    def fill_deg(k, _):
        degbuf[pl.ds(k * 16, 16)] = jnp.full((16,), 1.0, jnp.float32)
        return _
    lax.fori_loop(0, ROWS_PER_TILE // 16, fill_deg, None)
    pltpu.sync_copy(degbuf, deg_sh.at[pl.ds(base, ROWS_PER_TILE)])

    def zero_row(r, _):
        for cc in range(D // 16):
            rows0[r, pl.ds(cc * 16, 16)] = jnp.zeros((16,), jnp.float32)
        return _
    lax.fori_loop(0, CHUNK, zero_row, None)
    for t in range(ROWS_PER_TILE // CHUNK):
        pltpu.sync_copy(rows0, p_sh.at[pl.ds(base + t * CHUNK, CHUNK)])
    plsc.subcore_barrier()

    n_groups = NCH // GRP

    def deg_group(i, _):
        blk = s + (i // n_groups) * NS
        g = i % n_groups
        pltpu.sync_copy(dstb.at[blk, pl.ds(g * GRP, GRP)], dst_v)
        pltpu.sync_copy(ewb.at[blk, pl.ds(g * GRP, GRP)], ew_v)

        def deg_chunk(j, _2):
            pltpu.sync_copy(ew_v.at[j], deg_sh.at[dst_v.at[j]], add=True)
            return _2
        lax.fori_loop(0, GRP, deg_chunk, None)
        return _
    lax.fori_loop(0, 2 * n_groups, deg_group, None)
    plsc.subcore_barrier()

    pltpu.sync_copy(deg_sh.at[pl.ds(base, ROWS_PER_TILE)], degbuf)

    def rs_step(k, _):
        sl = pl.ds(k * 16, 16)
        degbuf[sl] = _rsqrt16(degbuf[sl])
        return _
    lax.fori_loop(0, ROWS_PER_TILE // 16, rs_step, None)
    pltpu.sync_copy(degbuf, deg_sh.at[pl.ds(base, ROWS_PER_TILE)])

    @pl.when(c == 0)
    def _():
        pltpu.sync_copy(degbuf, dinv_out.at[pl.ds(base, ROWS_PER_TILE)])
    plsc.subcore_barrier()
    pltpu.sync_copy(deg_sh, dinv_v)

    def norm_scale(j, rows_b):
        for cc in range(CHUNK // 16):
            sl = pl.ds(cc * 16, 16)
            si = src_v[j, sl]
            di = dst_v[j, sl]
            ds_ = plsc.load_gather(dinv_v, [si])
            dd_ = plsc.load_gather(dinv_v, [di])
            norm_v[sl] = ds_ * ew_v[j, sl] * dd_

        def scale2(r2, _2):
            for rr in range(2):
                r = r2 * 2 + rr
                nsp = plsc.load_gather(norm_v, [jnp.full((16,), r, jnp.int32)])
                for cc in range(D // 16):
                    sl = pl.ds(cc * 16, 16)
                    rows_b[r, sl] = rows_b[r, sl] * nsp
            return _2
        lax.fori_loop(0, CHUNK // 2, scale2, None)

    def fire_gather(j, rows_b, sem):
        pltpu.make_async_copy(x_hbm.at[src_v.at[j]], rows_b, sem).start()

    def wait_gather(j, rows_b, sem):
        pltpu.make_async_copy(x_hbm.at[src_v.at[j]], rows_b, sem).wait()

    def scatter(j, rows_b):
        pltpu.sync_copy(rows_b, p_sh.at[dst_v.at[j]], add=True)

    def group_body(g, _):
        pltpu.sync_copy(srcb.at[w, pl.ds(g * GRP, GRP)], src_v)
        pltpu.sync_copy(dstb.at[w, pl.ds(g * GRP, GRP)], dst_v)
        pltpu.sync_copy(ewb.at[w, pl.ds(g * GRP, GRP)], ew_v)
        fire_gather(0, rows0, sg0)
        fire_gather(1, rows1, sg1)

        def pair_body(p, _2):
            j0 = 2 * p
            j1 = j0 + 1
            wait_gather(j0, rows0, sg0)
            norm_scale(j0, rows0)
            scatter(j0, rows0)

            @pl.when(p < GRP // 2 - 1)
            def _f0():
                fire_gather(j0 + 2, rows0, sg0)
            wait_gather(j1, rows1, sg1)
            norm_scale(j1, rows1)
            scatter(j1, rows1)

            @pl.when(p < GRP // 2 - 1)
            def _f1():
                fire_gather(j1 + 2, rows1, sg1)
            return _2
        lax.fori_loop(0, GRP // 2, pair_body, None)
        return _
    lax.fori_loop(0, NCH // GRP, group_body, None)
    plsc.subcore_barrier()

    pltpu.sync_copy(p_sh.at[pl.ds(base, ROWS_PER_TILE)],
                    pout.at[c, pl.ds(base, ROWS_PER_TILE)])


def _sc_propagate(x, srcb, dstb, ewb):
    mesh = plsc.VectorSubcoreMesh(core_axis_name="c", subcore_axis_name="s")
    f = pl.kernel(
        _sc_body,
        out_type=[
            jax.ShapeDtypeStruct((NC, NPAD, D), jnp.float32),
            jax.ShapeDtypeStruct((NPAD,), jnp.float32),
        ],
        mesh=mesh,
        scratch_types=[
            pltpu.VMEM((GRP, CHUNK), jnp.int32),
            pltpu.VMEM((GRP, CHUNK), jnp.int32),
            pltpu.VMEM((GRP, CHUNK), jnp.float32),
            pltpu.VMEM((NPAD,), jnp.float32),
            pltpu.VMEM((CHUNK, D), jnp.float32),
            pltpu.VMEM((CHUNK, D), jnp.float32),
            pltpu.VMEM((CHUNK,), jnp.float32),
            pltpu.VMEM((ROWS_PER_TILE,), jnp.float32),
            pltpu.VMEM_SHARED((NPAD, D), jnp.float32),
            pltpu.VMEM_SHARED((NPAD,), jnp.float32),
            pltpu.SemaphoreType.DMA,
            pltpu.SemaphoreType.DMA,
        ],
        compiler_params=pltpu.CompilerParams(needs_layout_passes=False),
    )
    return f(x, srcb, dstb, ewb)


def _dense_body(x_ref, p0_ref, p1_ref, dv_ref, wz_ref, bz_ref, wh_ref, bh_ref,
                lzw_ref, lzb_ref, lhw_ref, lhb_ref, hw_ref, hb_ref, o_ref):
    dv = dv_ref[...]
    P = p0_ref[...] + p1_ref[...] + (dv * dv) * x_ref[...]
    f32 = jnp.float32
    mz = jnp.dot(wz_ref[...], lzw_ref[...], preferred_element_type=f32)
    cz = jnp.dot(bz_ref[...], lzw_ref[...], preferred_element_type=f32) + lzb_ref[...]
    z = jax.nn.sigmoid(jnp.dot(P, mz, preferred_element_type=f32) + cz)
    mh = jnp.dot(wh_ref[...], lhw_ref[...], preferred_element_type=f32)
    ch = jnp.dot(bh_ref[...], lhw_ref[...], preferred_element_type=f32) + lhb_ref[...]
    ht = jnp.tanh(jnp.dot(P, mh, preferred_element_type=f32) + ch)
    o_ref[...] = jnp.dot((1.0 - z) * ht, hw_ref[...], preferred_element_type=f32) + hb_ref[...]


def _dense(xp, p0, p1, dv2, Wz, bz, Wh, bh, Lzw1, Lzb, Lhw1, Lhb, head_w, head_b):
    blk = 1280
    grid = (NPAD // blk,)
    row = lambda i: (i, 0)
    fix = lambda i: (0, 0)
    return pl.pallas_call(
        _dense_body,
        grid=grid,
        in_specs=[
            pl.BlockSpec((blk, D), row),
            pl.BlockSpec((blk, D), row),
            pl.BlockSpec((blk, D), row),
            pl.BlockSpec((blk, 1), row),
            pl.BlockSpec((D, D), fix),
            pl.BlockSpec((1, D), fix),
            pl.BlockSpec((D, D), fix),
            pl.BlockSpec((1, D), fix),
            pl.BlockSpec((D, D), fix),
            pl.BlockSpec((1, D), fix),
            pl.BlockSpec((D, D), fix),
            pl.BlockSpec((1, D), fix),
            pl.BlockSpec((D, 1), fix),
            pl.BlockSpec((1, 1), fix),
        ],
        out_specs=pl.BlockSpec((blk, 1), row),
        out_shape=jax.ShapeDtypeStruct((NPAD, 1), jnp.float32),
    )(xp, p0, p1, dv2, Wz, bz, Wh, bh, Lzw1, Lzb, Lhw1, Lhb, head_w, head_b)


def kernel(x, edge_index, edge_weight, Wz, bz, Wr, br, Wh, bh,
           Lz_w, Lz_b, Lr_w, Lr_b, Lh_w, Lh_b, head_w, head_b):
    E = edge_index.shape[1]
    pad = EPAD - E
    src = jnp.concatenate([edge_index[0], jnp.zeros((pad,), jnp.int32)])
    dst = jnp.concatenate([edge_index[1], jnp.zeros((pad,), jnp.int32)])
    ew = jnp.concatenate([edge_weight, jnp.zeros((pad,), jnp.float32)])
    srcb = src.reshape(NC * NS, NCH, CHUNK)
    dstb = dst.reshape(NC * NS, NCH, CHUNK)
    ewb = ew.reshape(NC * NS, NCH, CHUNK)

    pout, dinv = _sc_propagate(x, srcb, dstb, ewb)

    xp = jnp.zeros((NPAD, D), jnp.float32).at[:N_NODES].set(x)
    out = _dense(
        xp, pout[0], pout[1], dinv.reshape(NPAD, 1),
        Wz, bz[None, :], Wh, bh[None, :],
        Lz_w[:D], Lz_b[None, :], Lh_w[:D], Lh_b[None, :],
        head_w, head_b[None, :],
    )
    return out[:N_NODES, 0]
